# baseline (speedup 1.0000x reference)
import math
import functools

import jax
import jax.numpy as jnp
from jax import lax
from jax.experimental import pallas as pl
from jax.experimental.pallas import tpu as pltpu
from jax.experimental.pallas import tpu_sc as plsc

N = 10000
NP = 10240
PAD = NP - N
D = 128
E = 160000
CH = 128
CHUNKS = 80
EP = 16 * CH * CHUNKS
ET = CH * CHUNKS
BLK = 1024
GB = NP // BLK
E2 = math.exp(2.0)
INVT = 2.0

def _mesh():
    return plsc.VectorSubcoreMesh(core_axis_name="c", subcore_axis_name="s")


def _sc_deg_body(dst_hbm, zn_hbm, degp_hbm, dst_v, deg_v):
    c = lax.axis_index("c")
    s = lax.axis_index("s")
    wid = c * 16 + s
    pltpu.sync_copy(dst_hbm.at[pl.ds(wid * ET, ET)], dst_v)
    pltpu.sync_copy(zn_hbm, deg_v)
    ones = jnp.full((16,), 1.0, jnp.float32)

    def body(k, carry):
        base = k * 16
        idx = dst_v[pl.ds(base, 16)]
        plsc.addupdate_scatter(deg_v, [idx], ones)
        return carry

    lax.fori_loop(0, ET // 16, body, 0)
    pltpu.sync_copy(deg_v, degp_hbm.at[wid])


def _sc_deg(dst1d, znodes):
    k = pl.kernel(
        _sc_deg_body,
        mesh=_mesh(),
        out_type=jax.ShapeDtypeStruct((32, NP), jnp.float32),
        scratch_types=[
            pltpu.VMEM((ET,), jnp.int32),
            pltpu.VMEM((NP,), jnp.float32),
        ],
        compiler_params=pltpu.CompilerParams(needs_layout_passes=False),
    )
    return k(dst1d, znodes)


def _sc_scatter_body(g_hbm, src_hbm, dst_hbm, zr_hbm, acc_hbm,
                     src_v, dst_v, rows0, rows1, acc_s, sem0, sem1):
    c = lax.axis_index("c")
    s = lax.axis_index("s")
    wid = c * 16 + s
    rows_per_tile = NP // 16
    pltpu.sync_copy(zr_hbm, acc_s.at[pl.ds(s * rows_per_tile, rows_per_tile)])
    plsc.subcore_barrier()

    hc = CHUNKS // 2
    he = ET // 2

    def fire(j, buf, sem):
        pltpu.async_copy(g_hbm.at[src_v.at[pl.ds(j * CH, CH)]], buf, sem)

    def drain(buf, sem):
        pltpu.make_async_copy(g_hbm.at[pl.ds(0, CH)], buf, sem).wait()

    for p in range(2):
        pltpu.sync_copy(src_hbm.at[pl.ds(wid * ET + p * he, he)], src_v)
        pltpu.sync_copy(dst_hbm.at[pl.ds(wid * CHUNKS + p * hc, hc)], dst_v)
        fire(0, rows0, sem0)

        def body(t, carry):
            c0 = 2 * t
            fire(c0 + 1, rows1, sem1)
            drain(rows0, sem0)
            pltpu.sync_copy(rows0, acc_s.at[dst_v.at[c0]], add=True)

            @pl.when(c0 + 2 < hc)
            def _():
                fire(c0 + 2, rows0, sem0)

            drain(rows1, sem1)
            pltpu.sync_copy(rows1, acc_s.at[dst_v.at[c0 + 1]], add=True)
            return carry

        lax.fori_loop(0, hc // 2, body, 0)
    plsc.subcore_barrier()
    pltpu.sync_copy(acc_s.at[pl.ds(s * rows_per_tile, rows_per_tile)],
                    acc_hbm.at[pl.ds(c * NP + s * rows_per_tile, rows_per_tile)])


def _sc_scatter(g_flat, src1d, dst2d, zrows):
    k = pl.kernel(
        _sc_scatter_body,
        mesh=_mesh(),
        out_type=jax.ShapeDtypeStruct((2 * NP, D), jnp.float32),
        scratch_types=[
            pltpu.VMEM((ET // 2,), jnp.int32),
            pltpu.VMEM((CHUNKS // 2, CH), jnp.int32),
            pltpu.VMEM((CH, D), jnp.float32),
            pltpu.VMEM((CH, D), jnp.float32),
            pltpu.VMEM_SHARED((NP, D), jnp.float32),
            pltpu.SemaphoreType.DMA,
            pltpu.SemaphoreType.DMA,
        ],
        compiler_params=pltpu.CompilerParams(needs_layout_passes=False),
    )
    return k(g_flat, src1d, dst2d, zrows)


def _prep_body(feat_ref, w_ref, degp_ref, h_ref, g_ref):
    x = feat_ref[0]
    w = w_ref[...]
    deg = jnp.sum(degp_ref[0], axis=1, keepdims=True) + 1.0
    dinv = lax.rsqrt(deg)
    h = jnp.dot(x, w, preferred_element_type=jnp.float32)
    h_ref[0] = h
    g_ref[0] = h * dinv


def _tc_prep(featp, W_gcn, degp_pad):
    return pl.pallas_call(
        _prep_body,
        grid=(2, GB),
        in_specs=[
            pl.BlockSpec((1, BLK, D), lambda c, r: (c, r, 0)),
            pl.BlockSpec((D, D), lambda c, r: (0, 0)),
            pl.BlockSpec((1, BLK, D), lambda c, r: (c, r, 0)),
        ],
        out_specs=[
            pl.BlockSpec((1, BLK, D), lambda c, r: (c, r, 0)),
            pl.BlockSpec((1, BLK, D), lambda c, r: (c, r, 0)),
        ],
        out_shape=[
            jax.ShapeDtypeStruct((2, NP, D), jnp.float32),
            jax.ShapeDtypeStruct((2, NP, D), jnp.float32),
        ],
    )(featp, W_gcn, degp_pad)


def _epi_body(h_ref, acc_ref, degp_ref, bg_ref, w1_ref, b1_ref, w2_ref,
              b2_ref, out_ref):
    r = pl.program_id(1)
    deg = jnp.sum(degp_ref[0], axis=1, keepdims=True) + 1.0
    dinv = lax.rsqrt(deg)
    h = h_ref[0]
    acc = acc_ref[0]
    out = acc * dinv + h * (dinv * dinv) + bg_ref[...]
    z = jnp.dot(out, w1_ref[...], preferred_element_type=jnp.float32) + b1_ref[...]
    z = jnp.where(z > 0, z, jnp.exp(z) - 1.0)
    z = jnp.dot(z, w2_ref[...], preferred_element_type=jnp.float32) + b2_ref[...]
    nrm = jnp.sqrt(jnp.sum(z * z, axis=1, keepdims=True))
    z = z / jnp.maximum(nrm, 1e-12)
    rows = r * BLK + lax.broadcasted_iota(jnp.int32, (BLK, 1), 0)
    out_ref[0] = jnp.where(rows < N, z, 0.0)


def _tc_epi(h, acc, degp_pad, b_gcn2, fc1_W, fc1_b2, fc2_W, fc2_b2):
    return pl.pallas_call(
        _epi_body,
        grid=(2, GB),
        in_specs=[
            pl.BlockSpec((1, BLK, D), lambda c, r: (c, r, 0)),
            pl.BlockSpec((1, BLK, D), lambda c, r: (c, r, 0)),
            pl.BlockSpec((1, BLK, D), lambda c, r: (c, r, 0)),
            pl.BlockSpec((1, D), lambda c, r: (0, 0)),
            pl.BlockSpec((D, D), lambda c, r: (0, 0)),
            pl.BlockSpec((1, D), lambda c, r: (0, 0)),
            pl.BlockSpec((D, D), lambda c, r: (0, 0)),
            pl.BlockSpec((1, D), lambda c, r: (0, 0)),
        ],
        out_specs=pl.BlockSpec((1, BLK, D), lambda c, r: (c, r, 0)),
        out_shape=jax.ShapeDtypeStruct((2, NP, D), jnp.float32),
    )(h, acc, degp_pad, b_gcn2, fc1_W, fc1_b2, fc2_W, fc2_b2)


def _flash_body(ai_ref, bi_ref, aj_ref, bj_ref, x1_ref, rb_ref, cba_ref,
                diag_ref, ra_s, rb_s, cab_s, cba_s):
    i = pl.program_id(0)
    j = pl.program_id(1)
    ni = pl.num_programs(0)
    nj = pl.num_programs(1)
    ai = ai_ref[...]
    bi = bi_ref[...]
    aj = aj_ref[...]
    bj = bj_ref[...]
    dn = (((1,), (1,)), ((), ()))

    def bexp(s):
        return jnp.exp((s * INVT).astype(jnp.bfloat16))

    eaa = bexp(lax.dot_general(aj, ai, dn, preferred_element_type=jnp.float32))
    ebb = bexp(lax.dot_general(bi, bj, dn, preferred_element_type=jnp.float32))
    eab = bexp(lax.dot_general(bj, ai, dn, preferred_element_type=jnp.float32))

    vaa = jnp.sum(eaa, axis=0, keepdims=True,
                  dtype=jnp.float32)
    vbb = jnp.sum(ebb, axis=1, keepdims=True,
                  dtype=jnp.float32)
    vab = jnp.sum(eab, axis=0, keepdims=True,
                  dtype=jnp.float32)
    vba = jnp.sum(eab, axis=1, keepdims=True,
                  dtype=jnp.float32)

    @pl.when(j == 0)
    def _():
        ra_s[...] = vaa
        rb_s[...] = vbb
        cab_s[...] = vab

    @pl.when(j > 0)
    def _():
        ra_s[...] = ra_s[...] + vaa
        rb_s[...] = rb_s[...] + vbb
        cab_s[...] = cab_s[...] + vab

    @pl.when(i == 0)
    def _():
        cba_s[pl.ds(j * BLK, BLK), :] = vba

    @pl.when(i > 0)
    def _():
        cba_s[pl.ds(j * BLK, BLK), :] = cba_s[pl.ds(j * BLK, BLK), :] + vba

    @pl.when(j == nj - 1)
    def _():
        x1_ref[...] = jnp.reshape(ra_s[...] + cab_s[...], (1, 1, BLK))
        rb_ref[...] = rb_s[...]
        diag_ref[...] = jnp.sum(ai.astype(jnp.float32) *
                                bi.astype(jnp.float32), axis=1, keepdims=True)

    @pl.when(i == ni - 1)
    def _():
        cba_ref[...] = cba_s[pl.ds(j * BLK, BLK), :]


def _tc_flash(a, b):
    return pl.pallas_call(
        _flash_body,
        grid=(GB, GB),
        in_specs=[
            pl.BlockSpec((BLK, D), lambda i, j: (i, 0)),
            pl.BlockSpec((BLK, D), lambda i, j: (i, 0)),
            pl.BlockSpec((BLK, D), lambda i, j: (j, 0)),
            pl.BlockSpec((BLK, D), lambda i, j: (j, 0)),
        ],
        out_specs=[
            pl.BlockSpec((1, 1, BLK), lambda i, j: (i, 0, 0)),
            pl.BlockSpec((BLK, 1), lambda i, j: (i, 0)),
            pl.BlockSpec((BLK, 1), lambda i, j: (j, 0)),
            pl.BlockSpec((BLK, 1), lambda i, j: (i, 0)),
        ],
        out_shape=[
            jax.ShapeDtypeStruct((GB, 1, BLK), jnp.float32),
            jax.ShapeDtypeStruct((NP, 1), jnp.float32),
            jax.ShapeDtypeStruct((NP, 1), jnp.float32),
            jax.ShapeDtypeStruct((NP, 1), jnp.float32),
        ],
        scratch_shapes=[
            pltpu.VMEM((1, BLK), jnp.float32),
            pltpu.VMEM((BLK, 1), jnp.float32),
            pltpu.VMEM((1, BLK), jnp.float32),
            pltpu.VMEM((NP, 1), jnp.float32),
        ],
    )(a, b, a, b)


def _asm_body(x1_ref, rb_ref, cba_ref, diag_ref, out_ref):
    corr = 2.0 * PAD + E2
    x1 = x1_ref[...] - corr
    node1 = (lax.broadcasted_iota(jnp.int32, (GB, 1, BLK), 0) * BLK +
             lax.broadcasted_iota(jnp.int32, (GB, 1, BLK), 2))
    t1 = jnp.sum(jnp.where(node1 < N, jnp.log(x1), 0.0))
    x2 = rb_ref[...] + cba_ref[...] - corr
    node2 = lax.broadcasted_iota(jnp.int32, (NP, 1), 0)
    mask2 = node2 < N
    t2 = jnp.sum(jnp.where(mask2, jnp.log(x2), 0.0))
    t3 = jnp.sum(jnp.where(mask2, diag_ref[...], 0.0))
    out_ref[...] = jnp.reshape(
        (0.5 * (t1 + t2) - INVT * t3) / N, (1, 1))


def _tc_asm(x1, rb, cba, diag):
    return pl.pallas_call(
        _asm_body,
        out_shape=jax.ShapeDtypeStruct((1, 1), jnp.float32),
    )(x1, rb, cba, diag)


def _pad_edges(e):
    src = jnp.concatenate([e[0], jnp.full((EP - E,), N, jnp.int32)])
    dst = jnp.concatenate([e[1], jnp.full((EP - E,), N, jnp.int32)])
    return src, dst


def kernel(edge1, edge2, feat1, feat2, W_gcn, b_gcn, fc1_W, fc1_b, fc2_W,
           fc2_b):
    src1, dst1 = _pad_edges(edge1)
    src2, dst2 = _pad_edges(edge2)
    dst1d = jnp.concatenate([dst1, dst2])
    src1d = jnp.concatenate([src1, src2 + NP])
    dst2d = dst1d.reshape(2 * 16 * CHUNKS, CH)
    znodes = jnp.zeros((NP,), jnp.float32)
    zrows = jnp.zeros((NP // 16, D), jnp.float32)

    degp = _sc_deg(dst1d, znodes)
    degp_pad = jnp.pad(
        degp.reshape(2, 16, NP).transpose(0, 2, 1), ((0, 0), (0, 0), (0, 112)))

    featp = jnp.stack([
        jnp.pad(feat1, ((0, PAD), (0, 0))),
        jnp.pad(feat2, ((0, PAD), (0, 0))),
    ])
    h, g = _tc_prep(featp, W_gcn, degp_pad)

    acc_flat = _sc_scatter(g.reshape(2 * NP, D), src1d, dst2d, zrows)
    acc = acc_flat.reshape(2, NP, D)

    anorm = _tc_epi(h, acc, degp_pad, b_gcn.reshape(1, D), fc1_W,
                    fc1_b.reshape(1, D), fc2_W, fc2_b.reshape(1, D))

    x1, rb, cba, diag = _tc_flash(anorm[0].astype(jnp.bfloat16),
                                  anorm[1].astype(jnp.bfloat16))
    loss = _tc_asm(x1, rb, cba, diag)
    return loss[0, 0]


# speedup vs baseline: 1.0717x; 1.0717x over previous
import math
import functools

import jax
import jax.numpy as jnp
from jax import lax
from jax.experimental import pallas as pl
from jax.experimental.pallas import tpu as pltpu
from jax.experimental.pallas import tpu_sc as plsc

N = 10000
NP = 10240
PAD = NP - N
D = 128
E = 160000
CH = 128
CHUNKS = 80
EP = 16 * CH * CHUNKS
ET = CH * CHUNKS
BLK = 1024
GB = NP // BLK
E2 = math.exp(2.0)
INVT = 2.0

def _mesh():
    return plsc.VectorSubcoreMesh(core_axis_name="c", subcore_axis_name="s")


def _sc_deg_body(dst_hbm, zn_hbm, degp_hbm, dst_v, deg_v):
    c = lax.axis_index("c")
    s = lax.axis_index("s")
    wid = c * 16 + s
    pltpu.sync_copy(dst_hbm.at[pl.ds(wid * ET, ET)], dst_v)
    pltpu.sync_copy(zn_hbm, deg_v)
    ones = jnp.full((16,), 1.0, jnp.float32)

    def body(k, carry):
        base = k * 16
        idx = dst_v[pl.ds(base, 16)]
        plsc.addupdate_scatter(deg_v, [idx], ones)
        return carry

    lax.fori_loop(0, ET // 16, body, 0)
    pltpu.sync_copy(deg_v, degp_hbm.at[wid])


def _sc_deg(dst1d, znodes):
    k = pl.kernel(
        _sc_deg_body,
        mesh=_mesh(),
        out_type=jax.ShapeDtypeStruct((32, NP), jnp.float32),
        scratch_types=[
            pltpu.VMEM((ET,), jnp.int32),
            pltpu.VMEM((NP,), jnp.float32),
        ],
        compiler_params=pltpu.CompilerParams(needs_layout_passes=False),
    )
    return k(dst1d, znodes)


def _sc_scatter_body(g_hbm, src_hbm, dst_hbm, zr_hbm, acc_hbm,
                     src_v, dst_v, rows0, rows1, acc_s, sem0, sem1):
    c = lax.axis_index("c")
    s = lax.axis_index("s")
    wid = c * 16 + s
    rows_per_tile = NP // 16
    pltpu.sync_copy(zr_hbm, acc_s.at[pl.ds(s * rows_per_tile, rows_per_tile)])
    plsc.subcore_barrier()

    hc = CHUNKS // 2
    he = ET // 2

    def fire(j, buf, sem):
        pltpu.async_copy(g_hbm.at[src_v.at[pl.ds(j * CH, CH)]], buf, sem)

    def drain(buf, sem):
        pltpu.make_async_copy(g_hbm.at[pl.ds(0, CH)], buf, sem).wait()

    for p in range(2):
        pltpu.sync_copy(src_hbm.at[pl.ds(wid * ET + p * he, he)], src_v)
        pltpu.sync_copy(dst_hbm.at[pl.ds(wid * CHUNKS + p * hc, hc)], dst_v)
        fire(0, rows0, sem0)

        def body(t, carry):
            c0 = 2 * t
            fire(c0 + 1, rows1, sem1)
            drain(rows0, sem0)
            pltpu.sync_copy(rows0, acc_s.at[dst_v.at[c0]], add=True)

            @pl.when(c0 + 2 < hc)
            def _():
                fire(c0 + 2, rows0, sem0)

            drain(rows1, sem1)
            pltpu.sync_copy(rows1, acc_s.at[dst_v.at[c0 + 1]], add=True)
            return carry

        lax.fori_loop(0, hc // 2, body, 0)
    plsc.subcore_barrier()
    pltpu.sync_copy(acc_s.at[pl.ds(s * rows_per_tile, rows_per_tile)],
                    acc_hbm.at[pl.ds(c * NP + s * rows_per_tile, rows_per_tile)])


def _sc_scatter(g_flat, src1d, dst2d, zrows):
    k = pl.kernel(
        _sc_scatter_body,
        mesh=_mesh(),
        out_type=jax.ShapeDtypeStruct((2 * NP, D), jnp.float32),
        scratch_types=[
            pltpu.VMEM((ET // 2,), jnp.int32),
            pltpu.VMEM((CHUNKS // 2, CH), jnp.int32),
            pltpu.VMEM((CH, D), jnp.float32),
            pltpu.VMEM((CH, D), jnp.float32),
            pltpu.VMEM_SHARED((NP, D), jnp.float32),
            pltpu.SemaphoreType.DMA,
            pltpu.SemaphoreType.DMA,
        ],
        compiler_params=pltpu.CompilerParams(needs_layout_passes=False),
    )
    return k(g_flat, src1d, dst2d, zrows)


def _prep_body(feat_ref, w_ref, degp_ref, h_ref, g_ref):
    x = feat_ref[0]
    w = w_ref[...]
    deg = jnp.sum(degp_ref[0], axis=1, keepdims=True) + 1.0
    dinv = lax.rsqrt(deg)
    h = jnp.dot(x, w, preferred_element_type=jnp.float32)
    h_ref[0] = h
    g_ref[0] = h * dinv


def _tc_prep(featp, W_gcn, degp_pad):
    return pl.pallas_call(
        _prep_body,
        grid=(2, GB),
        in_specs=[
            pl.BlockSpec((1, BLK, D), lambda c, r: (c, r, 0)),
            pl.BlockSpec((D, D), lambda c, r: (0, 0)),
            pl.BlockSpec((1, BLK, D), lambda c, r: (c, r, 0)),
        ],
        out_specs=[
            pl.BlockSpec((1, BLK, D), lambda c, r: (c, r, 0)),
            pl.BlockSpec((1, BLK, D), lambda c, r: (c, r, 0)),
        ],
        out_shape=[
            jax.ShapeDtypeStruct((2, NP, D), jnp.float32),
            jax.ShapeDtypeStruct((2, NP, D), jnp.float32),
        ],
    )(featp, W_gcn, degp_pad)


def _epi_body(h_ref, acc_ref, degp_ref, bg_ref, w1_ref, b1_ref, w2_ref,
              b2_ref, out_ref):
    r = pl.program_id(1)
    deg = jnp.sum(degp_ref[0], axis=1, keepdims=True) + 1.0
    dinv = lax.rsqrt(deg)
    h = h_ref[0]
    acc = acc_ref[0]
    out = acc * dinv + h * (dinv * dinv) + bg_ref[...]
    z = jnp.dot(out, w1_ref[...], preferred_element_type=jnp.float32) + b1_ref[...]
    z = jnp.where(z > 0, z, jnp.exp(z) - 1.0)
    z = jnp.dot(z, w2_ref[...], preferred_element_type=jnp.float32) + b2_ref[...]
    nrm = jnp.sqrt(jnp.sum(z * z, axis=1, keepdims=True))
    z = z / jnp.maximum(nrm, 1e-12)
    rows = r * BLK + lax.broadcasted_iota(jnp.int32, (BLK, 1), 0)
    out_ref[0] = jnp.where(rows < N, z, 0.0)


def _tc_epi(h, acc, degp_pad, b_gcn2, fc1_W, fc1_b2, fc2_W, fc2_b2):
    return pl.pallas_call(
        _epi_body,
        grid=(2, GB),
        in_specs=[
            pl.BlockSpec((1, BLK, D), lambda c, r: (c, r, 0)),
            pl.BlockSpec((1, BLK, D), lambda c, r: (c, r, 0)),
            pl.BlockSpec((1, BLK, D), lambda c, r: (c, r, 0)),
            pl.BlockSpec((1, D), lambda c, r: (0, 0)),
            pl.BlockSpec((D, D), lambda c, r: (0, 0)),
            pl.BlockSpec((1, D), lambda c, r: (0, 0)),
            pl.BlockSpec((D, D), lambda c, r: (0, 0)),
            pl.BlockSpec((1, D), lambda c, r: (0, 0)),
        ],
        out_specs=pl.BlockSpec((1, BLK, D), lambda c, r: (c, r, 0)),
        out_shape=jax.ShapeDtypeStruct((2, NP, D), jnp.float32),
    )(h, acc, degp_pad, b_gcn2, fc1_W, fc1_b2, fc2_W, fc2_b2)


def _flash_body(ai_ref, bi_ref, aj_ref, bj_ref, x1_ref, rb_ref, cba_ref,
                diag_ref, ra_s, rb_s, cab_s, cba_s):
    i = pl.program_id(0)
    j = pl.program_id(1)
    ni = pl.num_programs(0)
    nj = pl.num_programs(1)
    ai = ai_ref[...]
    bi = bi_ref[...]
    aj = aj_ref[...]
    bj = bj_ref[...]
    dn = (((1,), (1,)), ((), ()))

    def fexp(s):
        return jnp.exp(s * INVT)

    eaa = fexp(lax.dot_general(aj, ai, dn, preferred_element_type=jnp.float32))
    ebb = fexp(lax.dot_general(bi, bj, dn, preferred_element_type=jnp.float32))
    eab = fexp(lax.dot_general(bj, ai, dn, preferred_element_type=jnp.float32))

    vaa = jnp.sum(eaa, axis=0, keepdims=True,
                  dtype=jnp.float32)
    vbb = jnp.sum(ebb, axis=1, keepdims=True,
                  dtype=jnp.float32)
    vab = jnp.sum(eab, axis=0, keepdims=True,
                  dtype=jnp.float32)
    vba = jnp.sum(eab, axis=1, keepdims=True,
                  dtype=jnp.float32)

    @pl.when(j == 0)
    def _():
        ra_s[...] = vaa
        rb_s[...] = vbb
        cab_s[...] = vab

    @pl.when(j > 0)
    def _():
        ra_s[...] = ra_s[...] + vaa
        rb_s[...] = rb_s[...] + vbb
        cab_s[...] = cab_s[...] + vab

    @pl.when(i == 0)
    def _():
        cba_s[pl.ds(j * BLK, BLK), :] = vba

    @pl.when(i > 0)
    def _():
        cba_s[pl.ds(j * BLK, BLK), :] = cba_s[pl.ds(j * BLK, BLK), :] + vba

    @pl.when(j == nj - 1)
    def _():
        x1_ref[...] = jnp.reshape(ra_s[...] + cab_s[...], (1, 1, BLK))
        rb_ref[...] = rb_s[...]
        diag_ref[...] = jnp.sum(ai.astype(jnp.float32) *
                                bi.astype(jnp.float32), axis=1, keepdims=True)

    @pl.when(i == ni - 1)
    def _():
        cba_ref[...] = cba_s[pl.ds(j * BLK, BLK), :]


def _tc_flash(a, b):
    return pl.pallas_call(
        _flash_body,
        grid=(GB, GB),
        in_specs=[
            pl.BlockSpec((BLK, D), lambda i, j: (i, 0)),
            pl.BlockSpec((BLK, D), lambda i, j: (i, 0)),
            pl.BlockSpec((BLK, D), lambda i, j: (j, 0)),
            pl.BlockSpec((BLK, D), lambda i, j: (j, 0)),
        ],
        out_specs=[
            pl.BlockSpec((1, 1, BLK), lambda i, j: (i, 0, 0)),
            pl.BlockSpec((BLK, 1), lambda i, j: (i, 0)),
            pl.BlockSpec((BLK, 1), lambda i, j: (j, 0)),
            pl.BlockSpec((BLK, 1), lambda i, j: (i, 0)),
        ],
        out_shape=[
            jax.ShapeDtypeStruct((GB, 1, BLK), jnp.float32),
            jax.ShapeDtypeStruct((NP, 1), jnp.float32),
            jax.ShapeDtypeStruct((NP, 1), jnp.float32),
            jax.ShapeDtypeStruct((NP, 1), jnp.float32),
        ],
        scratch_shapes=[
            pltpu.VMEM((1, BLK), jnp.float32),
            pltpu.VMEM((BLK, 1), jnp.float32),
            pltpu.VMEM((1, BLK), jnp.float32),
            pltpu.VMEM((NP, 1), jnp.float32),
        ],
    )(a, b, a, b)


def _asm_body(x1_ref, rb_ref, cba_ref, diag_ref, out_ref):
    corr = 2.0 * PAD + E2
    x1 = x1_ref[...] - corr
    node1 = (lax.broadcasted_iota(jnp.int32, (GB, 1, BLK), 0) * BLK +
             lax.broadcasted_iota(jnp.int32, (GB, 1, BLK), 2))
    t1 = jnp.sum(jnp.where(node1 < N, jnp.log(x1), 0.0))
    x2 = rb_ref[...] + cba_ref[...] - corr
    node2 = lax.broadcasted_iota(jnp.int32, (NP, 1), 0)
    mask2 = node2 < N
    t2 = jnp.sum(jnp.where(mask2, jnp.log(x2), 0.0))
    t3 = jnp.sum(jnp.where(mask2, diag_ref[...], 0.0))
    out_ref[...] = jnp.reshape(
        (0.5 * (t1 + t2) - INVT * t3) / N, (1, 1))


def _tc_asm(x1, rb, cba, diag):
    return pl.pallas_call(
        _asm_body,
        out_shape=jax.ShapeDtypeStruct((1, 1), jnp.float32),
    )(x1, rb, cba, diag)


def _pad_edges(e):
    src = jnp.concatenate([e[0], jnp.full((EP - E,), N, jnp.int32)])
    dst = jnp.concatenate([e[1], jnp.full((EP - E,), N, jnp.int32)])
    return src, dst


def kernel(edge1, edge2, feat1, feat2, W_gcn, b_gcn, fc1_W, fc1_b, fc2_W,
           fc2_b):
    src1, dst1 = _pad_edges(edge1)
    src2, dst2 = _pad_edges(edge2)
    dst1d = jnp.concatenate([dst1, dst2])
    src1d = jnp.concatenate([src1, src2 + NP])
    dst2d = dst1d.reshape(2 * 16 * CHUNKS, CH)
    znodes = jnp.zeros((NP,), jnp.float32)
    zrows = jnp.zeros((NP // 16, D), jnp.float32)

    degp = _sc_deg(dst1d, znodes)
    degp_pad = jnp.pad(
        degp.reshape(2, 16, NP).transpose(0, 2, 1), ((0, 0), (0, 0), (0, 112)))

    featp = jnp.stack([
        jnp.pad(feat1, ((0, PAD), (0, 0))),
        jnp.pad(feat2, ((0, PAD), (0, 0))),
    ])
    h, g = _tc_prep(featp, W_gcn, degp_pad)

    acc_flat = _sc_scatter(g.reshape(2 * NP, D), src1d, dst2d, zrows)
    acc = acc_flat.reshape(2, NP, D)

    anorm = _tc_epi(h, acc, degp_pad, b_gcn.reshape(1, D), fc1_W,
                    fc1_b.reshape(1, D), fc2_W, fc2_b.reshape(1, D))

    x1, rb, cba, diag = _tc_flash(anorm[0].astype(jnp.bfloat16),
                                  anorm[1].astype(jnp.bfloat16))
    loss = _tc_asm(x1, rb, cba, diag)
    return loss[0, 0]
